# Initial kernel scaffold; baseline (speedup 1.0000x reference)
import jax
import jax.numpy as jnp
from jax.experimental import pallas as pl


def kernel(x, edge_index, W, att_src, att_dst, bias):
    raise NotImplementedError("write your pallas kernel here")


# speedup vs baseline: 66.2618x; 66.2618x over previous
import functools

import jax
import jax.numpy as jnp
from jax import lax
from jax.experimental import pallas as pl
from jax.experimental.pallas import tpu as pltpu
from jax.experimental.pallas import tpu_sc as plsc

N = 10000
E = 320000
D_IN = 128
HEADS = 8
D_HEAD = 16
B = 2

LANES = 16
ROW = D_IN + LANES
N_PAD = 10112
NC = 2
NS = 16
NW = NC * NS
CHUNK = 128
E_TOT = E + N
EDGES_PER_TILE = 10368
NCHUNK = EDGES_PER_TILE // CHUNK
E_PAD = EDGES_PER_TILE * NW

BN = 1264
NB = N_PAD // BN
STRIPE = N_PAD // NS


def _proj_body(x_ref, w_ref, as_ref, ad_ref, h_ref, asrc_ref, adst_ref,
               mxs_ref, mxd_ref):
    nb = pl.program_id(1)
    h = jnp.dot(x_ref[0], w_ref[...], preferred_element_type=jnp.float32)
    h_ref[0] = h
    asrc = jnp.dot(h, as_ref[...], preferred_element_type=jnp.float32)
    adst = jnp.dot(h, ad_ref[...], preferred_element_type=jnp.float32)
    asrc_ref[0] = asrc
    adst_ref[0] = adst

    @pl.when(nb == 0)
    def _():
        mxs_ref[...] = jnp.full((1, 8, LANES), -jnp.inf, jnp.float32)
        mxd_ref[...] = jnp.full((1, 8, LANES), -jnp.inf, jnp.float32)

    mxs_ref[...] = jnp.maximum(
        mxs_ref[...], jnp.max(asrc, axis=0)[None, None, :])
    mxd_ref[...] = jnp.maximum(
        mxd_ref[...], jnp.max(adst, axis=0)[None, None, :])


def _norm_body(s0m_ref, s1m_ref, s0d_ref, s1d_ref, eexp_ref, bias_ref, out_ref):
    msg = s0m_ref[0] + s1m_ref[0]
    den = s0d_ref[0] + s1d_ref[0]
    den_exp = jnp.dot(den, eexp_ref[...], preferred_element_type=jnp.float32)
    out_ref[0] = msg / (den_exp + 1e-16) + bias_ref[...]


def _splat(vec, lane):
    idx = jnp.full((LANES,), lane, dtype=jnp.int32)
    dnums = lax.GatherDimensionNumbers(
        offset_dims=(), collapsed_slice_dims=(0,), start_index_map=(0,))
    return lax.gather(vec, idx[:, None], dnums, (1,),
                      mode=lax.GatherScatterMode.PROMISE_IN_BOUNDS)


_sc_mesh = plsc.VectorSubcoreMesh(core_axis_name="c", subcore_axis_name="s")


@functools.partial(
    pl.kernel,
    mesh=_sc_mesh,
    compiler_params=pltpu.CompilerParams(use_tc_tiling_on_sc=False),
    out_type=jax.ShapeDtypeStruct((B, NC, N_PAD, ROW), jnp.float32),
    scratch_types=[
        pltpu.VMEM((1, CHUNK), jnp.int32),
        pltpu.VMEM((1, CHUNK), jnp.int32),
        pltpu.VMEM((CHUNK, ROW), jnp.float32),
        pltpu.VMEM((CHUNK, LANES), jnp.float32),
        pltpu.VMEM((CHUNK, ROW), jnp.float32),
        pltpu.VMEM((LANES,), jnp.float32),
        pltpu.VMEM_SHARED((N_PAD, ROW), jnp.float32),
        pltpu.SemaphoreType.DMA,
    ],
)
def _edge_kernel(t1_hbm, t2_hbm, isrc_hbm, idst_hbm, c_hbm, zeros_hbm,
                 sout_hbm, ibs, ibd, t1buf, t2buf, msgbuf, cbuf, s_sh, sem):
    cid = lax.axis_index("c")
    sid = lax.axis_index("s")
    wid = sid * NC + cid

    for b in range(B):
        pltpu.sync_copy(zeros_hbm.at[pl.ds(sid * STRIPE, STRIPE)],
                        s_sh.at[pl.ds(sid * STRIPE, STRIPE)])
        pltpu.sync_copy(c_hbm.at[b], cbuf)
        plsc.subcore_barrier()
        cvec = cbuf[...]

        def chunk(j, carry):
            pltpu.sync_copy(isrc_hbm.at[wid, pl.ds(j, 1)], ibs)
            pltpu.sync_copy(idst_hbm.at[wid, pl.ds(j, 1)], ibd)
            cp1 = pltpu.async_copy(t1_hbm.at[b].at[ibs.at[0]], t1buf, sem)
            cp2 = pltpu.async_copy(t2_hbm.at[b].at[ibd.at[0]], t2buf, sem)
            cp1.wait()
            cp2.wait()

            def edge(e, icarry):
                asv = t1buf[e, pl.ds(D_IN, LANES)]
                adv = t2buf[e, :]
                al = asv + adv
                al = jnp.where(al >= 0.0, al, 0.2 * al)
                ex = jnp.exp(al - cvec)
                msgbuf[e, pl.ds(D_IN, LANES)] = ex
                for hd in range(HEADS):
                    sp = _splat(ex, hd)
                    msgbuf[e, pl.ds(hd * LANES, LANES)] = (
                        t1buf[e, pl.ds(hd * LANES, LANES)] * sp)
                return icarry

            lax.fori_loop(0, CHUNK, edge, 0)
            pltpu.sync_copy(msgbuf, s_sh.at[ibd.at[0]], add=True)
            return carry

        lax.fori_loop(0, NCHUNK, chunk, 0)
        plsc.subcore_barrier()
        pltpu.sync_copy(
            s_sh.at[pl.ds(sid * STRIPE, STRIPE)],
            sout_hbm.at[b, cid, pl.ds(sid * STRIPE, STRIPE)])


def kernel(x, edge_index, W, att_src, att_dst, bias):
    f32 = jnp.float32
    x_pad = jnp.pad(x, ((0, 0), (0, N_PAD - N), (0, 0)))

    eye = jnp.eye(HEADS, dtype=f32)
    As = (eye[:, None, :] * att_src[:, :, None]).reshape(HEADS * D_HEAD, HEADS)
    As = jnp.pad(As, ((0, 0), (0, LANES - HEADS)))
    Ad = (eye[:, None, :] * att_dst[:, :, None]).reshape(HEADS * D_HEAD, HEADS)
    Ad = jnp.pad(Ad, ((0, 0), (0, LANES - HEADS)))

    h, asrc, adst, mxs, mxd = pl.pallas_call(
        _proj_body,
        grid=(B, NB),
        in_specs=[
            pl.BlockSpec((1, BN, D_IN), lambda b, nb: (b, nb, 0)),
            pl.BlockSpec((D_IN, D_IN), lambda b, nb: (0, 0)),
            pl.BlockSpec((D_IN, LANES), lambda b, nb: (0, 0)),
            pl.BlockSpec((D_IN, LANES), lambda b, nb: (0, 0)),
        ],
        out_specs=[
            pl.BlockSpec((1, BN, D_IN), lambda b, nb: (b, nb, 0)),
            pl.BlockSpec((1, BN, LANES), lambda b, nb: (b, nb, 0)),
            pl.BlockSpec((1, BN, LANES), lambda b, nb: (b, nb, 0)),
            pl.BlockSpec((1, 8, LANES), lambda b, nb: (b, 0, 0)),
            pl.BlockSpec((1, 8, LANES), lambda b, nb: (b, 0, 0)),
        ],
        out_shape=[
            jax.ShapeDtypeStruct((B, N_PAD, D_IN), f32),
            jax.ShapeDtypeStruct((B, N_PAD, LANES), f32),
            jax.ShapeDtypeStruct((B, N_PAD, LANES), f32),
            jax.ShapeDtypeStruct((B, 8, LANES), f32),
            jax.ShapeDtypeStruct((B, 8, LANES), f32),
        ],
    )(x_pad, W, As, Ad)

    cv = mxs[:, 0, :] + mxd[:, 0, :]
    cv = jnp.where(cv >= 0.0, cv, 0.2 * cv)

    t1 = jnp.concatenate([h, asrc], axis=-1)
    t2 = adst

    sl = jnp.arange(N, dtype=jnp.int32)
    pad = jnp.full((E_PAD - E_TOT,), N, jnp.int32)
    src = jnp.concatenate([edge_index[0], sl, pad]).reshape(NW, NCHUNK, CHUNK)
    dst = jnp.concatenate([edge_index[1], sl, pad]).reshape(NW, NCHUNK, CHUNK)
    zeros = jnp.zeros((N_PAD, ROW), f32)

    sout = _edge_kernel(t1, t2, src, dst, cv, zeros)

    e8 = jnp.repeat(jnp.eye(HEADS, dtype=f32), D_HEAD, axis=1)
    eexp = jnp.concatenate([e8, jnp.zeros((LANES - HEADS, D_IN), f32)])

    out = pl.pallas_call(
        _norm_body,
        grid=(B, NB),
        in_specs=[
            pl.BlockSpec((1, BN, D_IN), lambda b, nb: (b, nb, 0)),
            pl.BlockSpec((1, BN, D_IN), lambda b, nb: (b, nb, 0)),
            pl.BlockSpec((1, BN, LANES), lambda b, nb: (b, nb, 0)),
            pl.BlockSpec((1, BN, LANES), lambda b, nb: (b, nb, 0)),
            pl.BlockSpec((LANES, D_IN), lambda b, nb: (0, 0)),
            pl.BlockSpec((1, D_IN), lambda b, nb: (0, 0)),
        ],
        out_specs=pl.BlockSpec((1, BN, D_IN), lambda b, nb: (b, nb, 0)),
        out_shape=jax.ShapeDtypeStruct((B, N_PAD, D_IN), f32),
    )(sout[:, 0, :, :D_IN], sout[:, 1, :, :D_IN],
      sout[:, 0, :, D_IN:], sout[:, 1, :, D_IN:],
      eexp, bias.reshape(1, D_IN))

    return out[:, :N, :]


# speedup vs baseline: 94.1834x; 1.4214x over previous
import functools

import jax
import jax.numpy as jnp
from jax import lax
from jax.experimental import pallas as pl
from jax.experimental.pallas import tpu as pltpu
from jax.experimental.pallas import tpu_sc as plsc

N = 10000
E = 320000
D_IN = 128
HEADS = 8
D_HEAD = 16
B = 2

LANES = 16
ROW = D_IN + LANES
N_PAD = 10112
NC = 2
NS = 16
NW = NC * NS
CHUNK = 128
IG = 3
E_TOT = E + N
EDGES_PER_TILE = 10368
NCHUNK = EDGES_PER_TILE // CHUNK
E_PAD = EDGES_PER_TILE * NW

BN = 1264
NB = N_PAD // BN
STRIPE = N_PAD // NS


def _proj_body(x_ref, w_ref, as_ref, ad_ref, h_ref, asrc_ref, adst_ref,
               mxs_ref, mxd_ref):
    nb = pl.program_id(1)
    h = jnp.dot(x_ref[0], w_ref[...], preferred_element_type=jnp.float32)
    h_ref[0] = h
    asrc = jnp.dot(h, as_ref[...], preferred_element_type=jnp.float32)
    adst = jnp.dot(h, ad_ref[...], preferred_element_type=jnp.float32)
    asrc_ref[0] = asrc
    adst_ref[0] = adst

    @pl.when(nb == 0)
    def _():
        mxs_ref[...] = jnp.full((1, 8, LANES), -jnp.inf, jnp.float32)
        mxd_ref[...] = jnp.full((1, 8, LANES), -jnp.inf, jnp.float32)

    mxs_ref[...] = jnp.maximum(
        mxs_ref[...], jnp.max(asrc, axis=0)[None, None, :])
    mxd_ref[...] = jnp.maximum(
        mxd_ref[...], jnp.max(adst, axis=0)[None, None, :])


def _norm_body(s0m_ref, s1m_ref, s0d_ref, s1d_ref, eexp_ref, bias_ref, out_ref):
    msg = s0m_ref[0] + s1m_ref[0]
    den = s0d_ref[0] + s1d_ref[0]
    den_exp = jnp.dot(den, eexp_ref[...], preferred_element_type=jnp.float32)
    out_ref[0] = msg / (den_exp + 1e-16) + bias_ref[...]


def _splat(vec, lane):
    idx = jnp.full((LANES,), lane, dtype=jnp.int32)
    dnums = lax.GatherDimensionNumbers(
        offset_dims=(), collapsed_slice_dims=(0,), start_index_map=(0,))
    return lax.gather(vec, idx[:, None], dnums, (1,),
                      mode=lax.GatherScatterMode.PROMISE_IN_BOUNDS)


_sc_mesh = plsc.VectorSubcoreMesh(core_axis_name="c", subcore_axis_name="s")


@functools.partial(
    pl.kernel,
    mesh=_sc_mesh,
    compiler_params=pltpu.CompilerParams(use_tc_tiling_on_sc=False),
    out_type=jax.ShapeDtypeStruct((B, NC, N_PAD, ROW), jnp.float32),
    scratch_types=[
        pltpu.VMEM((IG, CHUNK), jnp.int32),
        pltpu.VMEM((IG, CHUNK), jnp.int32),
        pltpu.VMEM((CHUNK, ROW), jnp.float32),
        pltpu.VMEM((CHUNK, LANES), jnp.float32),
        pltpu.VMEM((CHUNK, ROW), jnp.float32),
        pltpu.VMEM((LANES,), jnp.float32),
        pltpu.VMEM_SHARED((N_PAD, ROW), jnp.float32),
        pltpu.SemaphoreType.DMA,
        pltpu.SemaphoreType.DMA,
    ],
)
def _edge_kernel(t1_hbm, t2_hbm, isrc_hbm, idst_hbm, c_hbm, zeros_hbm,
                 sout_hbm, ibs, ibd, t1buf, t2buf, msgbuf, cbuf, s_sh,
                 sem, sem_sc):
    cid = lax.axis_index("c")
    sid = lax.axis_index("s")
    wid = sid * NC + cid

    for b in range(B):
        pltpu.sync_copy(zeros_hbm.at[pl.ds(sid * STRIPE, STRIPE)],
                        s_sh.at[pl.ds(sid * STRIPE, STRIPE)])
        pltpu.sync_copy(c_hbm.at[b], cbuf)
        plsc.subcore_barrier()
        cvec = cbuf[...]

        def chunk(j, carry):
            r = j % IG

            @pl.when(j > 0)
            def _():
                pltpu.make_async_copy(msgbuf, s_sh.at[ibd.at[0]], sem_sc).wait()

            @pl.when(r == 0)
            def _():
                g = j // IG
                pltpu.sync_copy(isrc_hbm.at[wid, pl.ds(g * IG, IG)], ibs)
                pltpu.sync_copy(idst_hbm.at[wid, pl.ds(g * IG, IG)], ibd)

            cp1 = pltpu.async_copy(t1_hbm.at[b].at[ibs.at[r]], t1buf, sem)
            cp2 = pltpu.async_copy(t2_hbm.at[b].at[ibd.at[r]], t2buf, sem)
            cp1.wait()
            cp2.wait()

            @plsc.parallel_loop(0, CHUNK, 1, unroll=4)
            def edge(e):
                asv = t1buf[e, pl.ds(D_IN, LANES)]
                adv = t2buf[e, :]
                al = asv + adv
                al = jnp.where(al >= 0.0, al, 0.2 * al)
                ex = jnp.exp(al - cvec)
                msgbuf[e, pl.ds(D_IN, LANES)] = ex
                for hd in range(HEADS):
                    sp = _splat(ex, hd)
                    msgbuf[e, pl.ds(hd * LANES, LANES)] = (
                        t1buf[e, pl.ds(hd * LANES, LANES)] * sp)

            pltpu.async_copy(msgbuf, s_sh.at[ibd.at[r]], sem_sc, add=True)
            return carry

        lax.fori_loop(0, NCHUNK, chunk, 0)
        pltpu.make_async_copy(msgbuf, s_sh.at[ibd.at[IG - 1]], sem_sc).wait()
        plsc.subcore_barrier()
        pltpu.sync_copy(
            s_sh.at[pl.ds(sid * STRIPE, STRIPE)],
            sout_hbm.at[b, cid, pl.ds(sid * STRIPE, STRIPE)])


def kernel(x, edge_index, W, att_src, att_dst, bias):
    f32 = jnp.float32
    x_pad = jnp.pad(x, ((0, 0), (0, N_PAD - N), (0, 0)))

    eye = jnp.eye(HEADS, dtype=f32)
    As = (eye[:, None, :] * att_src[:, :, None]).reshape(HEADS * D_HEAD, HEADS)
    As = jnp.pad(As, ((0, 0), (0, LANES - HEADS)))
    Ad = (eye[:, None, :] * att_dst[:, :, None]).reshape(HEADS * D_HEAD, HEADS)
    Ad = jnp.pad(Ad, ((0, 0), (0, LANES - HEADS)))

    h, asrc, adst, mxs, mxd = pl.pallas_call(
        _proj_body,
        grid=(B, NB),
        in_specs=[
            pl.BlockSpec((1, BN, D_IN), lambda b, nb: (b, nb, 0)),
            pl.BlockSpec((D_IN, D_IN), lambda b, nb: (0, 0)),
            pl.BlockSpec((D_IN, LANES), lambda b, nb: (0, 0)),
            pl.BlockSpec((D_IN, LANES), lambda b, nb: (0, 0)),
        ],
        out_specs=[
            pl.BlockSpec((1, BN, D_IN), lambda b, nb: (b, nb, 0)),
            pl.BlockSpec((1, BN, LANES), lambda b, nb: (b, nb, 0)),
            pl.BlockSpec((1, BN, LANES), lambda b, nb: (b, nb, 0)),
            pl.BlockSpec((1, 8, LANES), lambda b, nb: (b, 0, 0)),
            pl.BlockSpec((1, 8, LANES), lambda b, nb: (b, 0, 0)),
        ],
        out_shape=[
            jax.ShapeDtypeStruct((B, N_PAD, D_IN), f32),
            jax.ShapeDtypeStruct((B, N_PAD, LANES), f32),
            jax.ShapeDtypeStruct((B, N_PAD, LANES), f32),
            jax.ShapeDtypeStruct((B, 8, LANES), f32),
            jax.ShapeDtypeStruct((B, 8, LANES), f32),
        ],
    )(x_pad, W, As, Ad)

    cv = mxs[:, 0, :] + mxd[:, 0, :]
    cv = jnp.where(cv >= 0.0, cv, 0.2 * cv)

    t1 = jnp.concatenate([h, asrc], axis=-1)
    t2 = adst

    sl = jnp.arange(N, dtype=jnp.int32)
    pad = jnp.full((E_PAD - E_TOT,), N, jnp.int32)
    src = jnp.concatenate([edge_index[0], sl, pad]).reshape(NW, NCHUNK, CHUNK)
    dst = jnp.concatenate([edge_index[1], sl, pad]).reshape(NW, NCHUNK, CHUNK)
    zeros = jnp.zeros((N_PAD, ROW), f32)

    sout = _edge_kernel(t1, t2, src, dst, cv, zeros)

    e8 = jnp.repeat(jnp.eye(HEADS, dtype=f32), D_HEAD, axis=1)
    eexp = jnp.concatenate([e8, jnp.zeros((LANES - HEADS, D_IN), f32)])

    out = pl.pallas_call(
        _norm_body,
        grid=(B, NB),
        in_specs=[
            pl.BlockSpec((1, BN, D_IN), lambda b, nb: (b, nb, 0)),
            pl.BlockSpec((1, BN, D_IN), lambda b, nb: (b, nb, 0)),
            pl.BlockSpec((1, BN, LANES), lambda b, nb: (b, nb, 0)),
            pl.BlockSpec((1, BN, LANES), lambda b, nb: (b, nb, 0)),
            pl.BlockSpec((LANES, D_IN), lambda b, nb: (0, 0)),
            pl.BlockSpec((1, D_IN), lambda b, nb: (0, 0)),
        ],
        out_specs=pl.BlockSpec((1, BN, D_IN), lambda b, nb: (b, nb, 0)),
        out_shape=jax.ShapeDtypeStruct((B, N_PAD, D_IN), f32),
    )(sout[:, 0, :, :D_IN], sout[:, 1, :, :D_IN],
      sout[:, 0, :, D_IN:], sout[:, 1, :, D_IN:],
      eexp, bias.reshape(1, D_IN))

    return out[:, :N, :]


# speedup vs baseline: 96.7293x; 1.0270x over previous
import functools

import jax
import jax.numpy as jnp
from jax import lax
from jax.experimental import pallas as pl
from jax.experimental.pallas import tpu as pltpu
from jax.experimental.pallas import tpu_sc as plsc

N = 10000
E = 320000
D_IN = 128
HEADS = 8
D_HEAD = 16
B = 2

LANES = 16
ROW = D_IN + LANES
N_PAD = 10112
NC = 2
NS = 16
NW = NC * NS
CHUNK = 128
E_TOT = E + N
EDGES_PER_TILE = 10368
NCHUNK = EDGES_PER_TILE // CHUNK
E_PAD = EDGES_PER_TILE * NW

BN = 1264
NB = N_PAD // BN
STRIPE = N_PAD // NS


def _proj_body(x_ref, w_ref, as_ref, ad_ref, h_ref, asrc_ref, adst_ref,
               mxs_ref, mxd_ref):
    nb = pl.program_id(1)
    h = jnp.dot(x_ref[0], w_ref[...], preferred_element_type=jnp.float32)
    h_ref[0] = h
    asrc = jnp.dot(h, as_ref[...], preferred_element_type=jnp.float32)
    adst = jnp.dot(h, ad_ref[...], preferred_element_type=jnp.float32)
    asrc_ref[0] = asrc
    adst_ref[0] = adst

    @pl.when(nb == 0)
    def _():
        mxs_ref[...] = jnp.full((1, 8, LANES), -jnp.inf, jnp.float32)
        mxd_ref[...] = jnp.full((1, 8, LANES), -jnp.inf, jnp.float32)

    mxs_ref[...] = jnp.maximum(
        mxs_ref[...], jnp.max(asrc, axis=0)[None, None, :])
    mxd_ref[...] = jnp.maximum(
        mxd_ref[...], jnp.max(adst, axis=0)[None, None, :])


def _norm_body(s0m_ref, s1m_ref, s0d_ref, s1d_ref, eexp_ref, bias_ref, out_ref):
    msg = s0m_ref[0] + s1m_ref[0]
    den = s0d_ref[0] + s1d_ref[0]
    den_exp = jnp.dot(den, eexp_ref[...], preferred_element_type=jnp.float32)
    out_ref[0] = msg / (den_exp + 1e-16) + bias_ref[...]


def _splat(vec, lane):
    idx = jnp.full((LANES,), lane, dtype=jnp.int32)
    dnums = lax.GatherDimensionNumbers(
        offset_dims=(), collapsed_slice_dims=(0,), start_index_map=(0,))
    return lax.gather(vec, idx[:, None], dnums, (1,),
                      mode=lax.GatherScatterMode.PROMISE_IN_BOUNDS)


_sc_mesh = plsc.VectorSubcoreMesh(core_axis_name="c", subcore_axis_name="s")


@functools.partial(
    pl.kernel,
    mesh=_sc_mesh,
    compiler_params=pltpu.CompilerParams(use_tc_tiling_on_sc=False),
    out_type=jax.ShapeDtypeStruct((B, NC, N_PAD, ROW), jnp.float32),
    scratch_types=[
        pltpu.VMEM((2, CHUNK), jnp.int32),
        pltpu.VMEM((2, CHUNK), jnp.int32),
        pltpu.VMEM((CHUNK, ROW), jnp.float32),
        pltpu.VMEM((CHUNK, LANES), jnp.float32),
        pltpu.VMEM((CHUNK, ROW), jnp.float32),
        pltpu.VMEM((LANES,), jnp.float32),
        pltpu.VMEM_SHARED((N_PAD, ROW), jnp.float32),
        pltpu.SemaphoreType.DMA,
        pltpu.SemaphoreType.DMA,
    ],
)
def _edge_kernel(t1_hbm, t2_hbm, isrc_hbm, idst_hbm, c_hbm, zeros_hbm,
                 sout_hbm, ibs, ibd, t1buf, t2buf, msgbuf, cbuf, s_sh,
                 sem, sem_sc):
    cid = lax.axis_index("c")
    sid = lax.axis_index("s")
    wid = sid * NC + cid

    for b in range(B):
        pltpu.sync_copy(zeros_hbm.at[pl.ds(sid * STRIPE, STRIPE)],
                        s_sh.at[pl.ds(sid * STRIPE, STRIPE)])
        pltpu.sync_copy(c_hbm.at[b], cbuf)
        plsc.subcore_barrier()
        cvec = cbuf[...]

        pltpu.sync_copy(isrc_hbm.at[wid, pl.ds(0, 1)], ibs.at[pl.ds(0, 1)])
        pltpu.sync_copy(idst_hbm.at[wid, pl.ds(0, 1)], ibd.at[pl.ds(0, 1)])
        pltpu.async_copy(t1_hbm.at[b].at[ibs.at[0]], t1buf, sem)
        pltpu.async_copy(t2_hbm.at[b].at[ibd.at[0]], t2buf, sem)

        def chunk(j, carry):
            r = j % 2
            pltpu.make_async_copy(t1_hbm.at[b].at[ibs.at[r]], t1buf, sem).wait()
            pltpu.make_async_copy(t2_hbm.at[b].at[ibd.at[r]], t2buf, sem).wait()
            @pl.when(j > 0)
            def _():
                pltpu.make_async_copy(msgbuf, s_sh.at[ibd.at[0]], sem_sc).wait()

            @plsc.parallel_loop(0, CHUNK, 1, unroll=8)
            def edge(e):
                asv = t1buf[e, pl.ds(D_IN, LANES)]
                adv = t2buf[e, :]
                al = asv + adv
                al = jnp.where(al >= 0.0, al, 0.2 * al)
                ex = jnp.exp(al - cvec)
                msgbuf[e, pl.ds(D_IN, LANES)] = ex
                for hd in range(HEADS):
                    sp = _splat(ex, hd)
                    msgbuf[e, pl.ds(hd * LANES, LANES)] = (
                        t1buf[e, pl.ds(hd * LANES, LANES)] * sp)

            pltpu.async_copy(msgbuf, s_sh.at[ibd.at[r]], sem_sc, add=True)

            @pl.when(j < NCHUNK - 1)
            def _():
                nr = (j + 1) % 2
                pltpu.sync_copy(isrc_hbm.at[wid, pl.ds(j + 1, 1)],
                                ibs.at[pl.ds(nr, 1)])
                pltpu.sync_copy(idst_hbm.at[wid, pl.ds(j + 1, 1)],
                                ibd.at[pl.ds(nr, 1)])
                pltpu.async_copy(t1_hbm.at[b].at[ibs.at[nr]], t1buf, sem)
                pltpu.async_copy(t2_hbm.at[b].at[ibd.at[nr]], t2buf, sem)

            return carry

        lax.fori_loop(0, NCHUNK, chunk, 0)
        pltpu.make_async_copy(msgbuf, s_sh.at[ibd.at[0]], sem_sc).wait()
        plsc.subcore_barrier()
        pltpu.sync_copy(
            s_sh.at[pl.ds(sid * STRIPE, STRIPE)],
            sout_hbm.at[b, cid, pl.ds(sid * STRIPE, STRIPE)])


def kernel(x, edge_index, W, att_src, att_dst, bias):
    f32 = jnp.float32
    x_pad = jnp.pad(x, ((0, 0), (0, N_PAD - N), (0, 0)))

    eye = jnp.eye(HEADS, dtype=f32)
    As = (eye[:, None, :] * att_src[:, :, None]).reshape(HEADS * D_HEAD, HEADS)
    As = jnp.pad(As, ((0, 0), (0, LANES - HEADS)))
    Ad = (eye[:, None, :] * att_dst[:, :, None]).reshape(HEADS * D_HEAD, HEADS)
    Ad = jnp.pad(Ad, ((0, 0), (0, LANES - HEADS)))

    h, asrc, adst, mxs, mxd = pl.pallas_call(
        _proj_body,
        grid=(B, NB),
        in_specs=[
            pl.BlockSpec((1, BN, D_IN), lambda b, nb: (b, nb, 0)),
            pl.BlockSpec((D_IN, D_IN), lambda b, nb: (0, 0)),
            pl.BlockSpec((D_IN, LANES), lambda b, nb: (0, 0)),
            pl.BlockSpec((D_IN, LANES), lambda b, nb: (0, 0)),
        ],
        out_specs=[
            pl.BlockSpec((1, BN, D_IN), lambda b, nb: (b, nb, 0)),
            pl.BlockSpec((1, BN, LANES), lambda b, nb: (b, nb, 0)),
            pl.BlockSpec((1, BN, LANES), lambda b, nb: (b, nb, 0)),
            pl.BlockSpec((1, 8, LANES), lambda b, nb: (b, 0, 0)),
            pl.BlockSpec((1, 8, LANES), lambda b, nb: (b, 0, 0)),
        ],
        out_shape=[
            jax.ShapeDtypeStruct((B, N_PAD, D_IN), f32),
            jax.ShapeDtypeStruct((B, N_PAD, LANES), f32),
            jax.ShapeDtypeStruct((B, N_PAD, LANES), f32),
            jax.ShapeDtypeStruct((B, 8, LANES), f32),
            jax.ShapeDtypeStruct((B, 8, LANES), f32),
        ],
    )(x_pad, W, As, Ad)

    cv = mxs[:, 0, :] + mxd[:, 0, :]
    cv = jnp.where(cv >= 0.0, cv, 0.2 * cv)

    t1 = jnp.concatenate([h, asrc], axis=-1)
    t2 = adst

    sl = jnp.arange(N, dtype=jnp.int32)
    pad = jnp.full((E_PAD - E_TOT,), N, jnp.int32)
    src = jnp.concatenate([edge_index[0], sl, pad]).reshape(NW, NCHUNK, CHUNK)
    dst = jnp.concatenate([edge_index[1], sl, pad]).reshape(NW, NCHUNK, CHUNK)
    zeros = jnp.zeros((N_PAD, ROW), f32)

    sout = _edge_kernel(t1, t2, src, dst, cv, zeros)

    e8 = jnp.repeat(jnp.eye(HEADS, dtype=f32), D_HEAD, axis=1)
    eexp = jnp.concatenate([e8, jnp.zeros((LANES - HEADS, D_IN), f32)])

    out = pl.pallas_call(
        _norm_body,
        grid=(B, NB),
        in_specs=[
            pl.BlockSpec((1, BN, D_IN), lambda b, nb: (b, nb, 0)),
            pl.BlockSpec((1, BN, D_IN), lambda b, nb: (b, nb, 0)),
            pl.BlockSpec((1, BN, LANES), lambda b, nb: (b, nb, 0)),
            pl.BlockSpec((1, BN, LANES), lambda b, nb: (b, nb, 0)),
            pl.BlockSpec((LANES, D_IN), lambda b, nb: (0, 0)),
            pl.BlockSpec((1, D_IN), lambda b, nb: (0, 0)),
        ],
        out_specs=pl.BlockSpec((1, BN, D_IN), lambda b, nb: (b, nb, 0)),
        out_shape=jax.ShapeDtypeStruct((B, N_PAD, D_IN), f32),
    )(sout[:, 0, :, :D_IN], sout[:, 1, :, :D_IN],
      sout[:, 0, :, D_IN:], sout[:, 1, :, D_IN:],
      eexp, bias.reshape(1, D_IN))

    return out[:, :N, :]
